# baseline (speedup 1.0000x reference)
import jax
import jax.numpy as jnp
from jax.experimental import pallas as pl
from jax.experimental.pallas import tpu as pltpu


def _conv_stack(x, w1_ref, b1_ref, w2_ref, b2_ref, w3_ref, b3_ref,
                hp, wp, hv, kh, kw):
    cout = w1_ref.shape[-1]
    l = hv * wp
    acc = jnp.zeros((l, cout), jnp.float32)
    for i in range(kh):
        for j in range(kw):
            d = i * wp + j
            acc += jnp.dot(x[d : d + l], w1_ref[i * kw + j],
                           preferred_element_type=jnp.float32)
    h = jnp.maximum(acc + b1_ref[...], 0.0).astype(jnp.bfloat16)
    h = jnp.dot(h, w2_ref[...], preferred_element_type=jnp.float32)
    h = jnp.maximum(h + b2_ref[...], 0.0).astype(jnp.bfloat16)
    h = jnp.dot(h, w3_ref[...], preferred_element_type=jnp.float32)
    return jnp.maximum(h + b3_ref[...], 0.0).astype(jnp.bfloat16)


def _maxpool_3x3_s2_repad(h, hv, wp, hp2, wp2, opad):
    c = h.shape[-1]
    po = (hv - 3) // 2 + 1
    g = h.reshape(hv, wp, c)
    a = g[: 2 * po].reshape(po, 2, wp, c).max(axis=1)
    b = g[2 : 2 * po + 2].reshape(po, 2, wp, c)[:, 0]
    rows = jnp.maximum(a, b)
    c1 = rows[:, : 2 * po].reshape(po, po, 2, c).max(axis=2)
    c2 = rows[:, 2 : 2 * po + 2].reshape(po, po, 2, c)[:, :, 0]
    pooled = jnp.maximum(c1, c2)
    out = jnp.pad(pooled, ((opad, hp2 - po - opad),
                           (opad, wp2 - po - opad), (0, 0)))
    return out.reshape(hp2 * wp2, c)


def _nin_body(x_ref,
              w11_ref, b11_ref, w12_ref, b12_ref, w13_ref, b13_ref,
              w21_ref, b21_ref, w22_ref, b22_ref, w23_ref, b23_ref,
              w31_ref, b31_ref, w32_ref, b32_ref, w33_ref, b33_ref,
              w41_ref, b41_ref, w42_ref, b42_ref, w43_ref, b43_ref,
              o_ref):
    x = x_ref[0].reshape(57 * 56, 48)
    h = _conv_stack(x, w11_ref, b11_ref, w12_ref, b12_ref, w13_ref, b13_ref,
                    hp=57, wp=56, hv=54, kh=3, kw=3)
    x = _maxpool_3x3_s2_repad(h, hv=54, wp=56, hp2=31, wp2=32, opad=2)
    h = _conv_stack(x, w21_ref, b21_ref, w22_ref, b22_ref, w23_ref, b23_ref,
                    hp=31, wp=32, hv=26, kh=5, kw=5)
    x = _maxpool_3x3_s2_repad(h, hv=26, wp=32, hp2=15, wp2=16, opad=1)
    h = _conv_stack(x, w31_ref, b31_ref, w32_ref, b32_ref, w33_ref, b33_ref,
                    hp=15, wp=16, hv=12, kh=3, kw=3)
    x = _maxpool_3x3_s2_repad(h, hv=12, wp=16, hp2=8, wp2=8, opad=1)
    h = _conv_stack(x, w41_ref, b41_ref, w42_ref, b42_ref, w43_ref, b43_ref,
                    hp=8, wp=8, hv=5, kh=3, kw=3)
    g = h.reshape(5, 8, 128)[:, :5].astype(jnp.float32)
    o_ref[0] = g.sum(axis=0).sum(axis=0, keepdims=True) / 25.0


def _space_to_depth4(x_nchw):
    n = x_nchw.shape[0]
    x = x_nchw.astype(jnp.bfloat16)
    x = x.reshape(n, 3, 56, 4, 56, 4).transpose(0, 2, 4, 3, 5, 1)
    x = x.reshape(n, 56, 56, 48)
    return jnp.pad(x, ((0, 0), (0, 1), (0, 0), (0, 0)))


def _prep_w1_b1(w):
    w = w[:363].reshape(11, 11, 3, 128)
    w = jnp.pad(w, ((0, 1), (0, 1), (0, 0), (0, 0)))
    w = w.reshape(3, 4, 3, 4, 3, 128).transpose(0, 2, 1, 3, 4, 5)
    return w.reshape(9, 48, 128)


def kernel(x, b1_w1, b1_b1, b1_w2, b1_b2, b1_w3, b1_b3,
           b2_w1, b2_b1, b2_w2, b2_b2, b2_w3, b2_b3,
           b3_w1, b3_b1, b3_w2, b3_b2, b3_w3, b3_b3,
           b4_w1, b4_b1, b4_w2, b4_b2, b4_w3, b4_b3):
    n = x.shape[0]
    w11 = _prep_w1_b1(b1_w1)
    w21 = jnp.pad(b2_w1[:2400].reshape(25, 96, 256), ((0, 0), (0, 32), (0, 0)))
    w31 = b3_w1[:2304].reshape(9, 256, 384)
    w41 = b4_w1[:3456].reshape(9, 384, 128)

    x1 = _space_to_depth4(x)

    weights = [w11, b1_b1, b1_w2, b1_b2, b1_w3, b1_b3,
               w21, b2_b1, b2_w2, b2_b2, b2_w3, b2_b3,
               w31, b3_b1, b3_w2, b3_b2, b3_w3, b3_b3,
               w41, b4_b1, b4_w2, b4_b2, b4_w3, b4_b3]
    w_specs = [pl.BlockSpec(w.shape, lambda i, nd=w.ndim: (0,) * nd)
               for w in weights]

    out = pl.pallas_call(
        _nin_body,
        out_shape=jax.ShapeDtypeStruct((n, 1, 128), jnp.float32),
        grid=(n,),
        in_specs=[pl.BlockSpec((1, 57, 56, 48), lambda i: (i, 0, 0, 0))]
        + w_specs,
        out_specs=pl.BlockSpec((1, 1, 128), lambda i: (i, 0, 0)),
        compiler_params=pltpu.CompilerParams(
            dimension_semantics=("parallel",)),
    )(x1, *weights)
    return out[:, 0, :10]


# speedup vs baseline: 1.3159x; 1.3159x over previous
import functools

import jax
import jax.numpy as jnp
from jax.experimental import pallas as pl
from jax.experimental.pallas import tpu as pltpu


def _maxpool_3x3_s2(h, hv, wp):
    c = h.shape[-1]
    po = (hv - 3) // 2 + 1
    g = h.reshape(hv, wp, c)
    a = g[: 2 * po].reshape(po, 2, wp, c).max(axis=1)
    b = g[2 : 2 * po + 2].reshape(po, 2, wp, c)[:, 0]
    rows = jnp.maximum(a, b)
    c1 = rows[:, : 2 * po].reshape(po, po, 2, c).max(axis=2)
    c2 = rows[:, 2 : 2 * po + 2].reshape(po, po, 2, c)[:, :, 0]
    return jnp.maximum(c1, c2)


def _block_body(x_ref, w1_ref, b1_ref, w2_ref, b2_ref, w3_ref, b3_ref, o_ref,
                *, hp, wp, hv, kh, kw, pool, opad):
    cin = x_ref.shape[-1]
    cout = w1_ref.shape[-1]
    x = x_ref[0].reshape(hp * wp, cin)
    l = hv * wp
    patches = jnp.concatenate(
        [x[i * wp + j : i * wp + j + l]
         for i in range(kh) for j in range(kw)], axis=1)
    h = jnp.dot(patches, w1_ref[...], preferred_element_type=jnp.float32)
    h = jnp.maximum(h + b1_ref[...], 0.0).astype(jnp.bfloat16)
    h = jnp.dot(h, w2_ref[...], preferred_element_type=jnp.float32)
    h = jnp.maximum(h + b2_ref[...], 0.0).astype(jnp.bfloat16)
    h = jnp.dot(h, w3_ref[...], preferred_element_type=jnp.float32)
    h = jnp.maximum(h + b3_ref[...], 0.0).astype(jnp.bfloat16)
    if pool == "max":
        po = (hv - 3) // 2 + 1
        o_ref[0] = jnp.zeros(o_ref.shape[1:], o_ref.dtype)
        o_ref[0, opad : opad + po, opad : opad + po, :] = _maxpool_3x3_s2(
            h, hv, wp)
    else:
        g = h.reshape(hv, wp, cout)[:, :hv].astype(jnp.float32)
        o_ref[0] = g.sum(axis=0).sum(axis=0, keepdims=True) / (hv * hv)


def _run_block(x, w1, b1, w2, b2, w3, b3, *, hp, wp, hv, kh, kw, pool,
               oh=0, ow=0, opad=0):
    n, cin = x.shape[0], x.shape[-1]
    cout = w1.shape[-1]
    body = functools.partial(_block_body, hp=hp, wp=wp, hv=hv, kh=kh, kw=kw,
                             pool=pool, opad=opad)
    if pool == "max":
        out_shape = jax.ShapeDtypeStruct((n, oh, ow, cout), jnp.bfloat16)
        out_spec = pl.BlockSpec((1, oh, ow, cout), lambda i: (i, 0, 0, 0))
    else:
        out_shape = jax.ShapeDtypeStruct((n, 1, cout), jnp.float32)
        out_spec = pl.BlockSpec((1, 1, cout), lambda i: (i, 0, 0))
    return pl.pallas_call(
        body,
        out_shape=out_shape,
        grid=(n,),
        in_specs=[
            pl.BlockSpec((1, hp, wp, cin), lambda i: (i, 0, 0, 0)),
            pl.BlockSpec(w1.shape, lambda i: (0, 0)),
            pl.BlockSpec(b1.shape, lambda i: (0, 0)),
            pl.BlockSpec(w2.shape, lambda i: (0, 0)),
            pl.BlockSpec(b2.shape, lambda i: (0, 0)),
            pl.BlockSpec(w3.shape, lambda i: (0, 0)),
            pl.BlockSpec(b3.shape, lambda i: (0, 0)),
        ],
        out_specs=out_spec,
        compiler_params=pltpu.CompilerParams(
            dimension_semantics=("parallel",)),
    )(x, w1, b1, w2, b2, w3, b3)


def _space_to_depth4(x_nchw):
    n = x_nchw.shape[0]
    x = x_nchw.astype(jnp.bfloat16)
    x = x.reshape(n, 3, 56, 4, 56, 4).transpose(0, 2, 4, 3, 5, 1)
    x = x.reshape(n, 56, 56, 48)
    return jnp.pad(x, ((0, 0), (0, 1), (0, 0), (0, 80)))


def _prep_w1_b1(w):
    w = w[:363].reshape(11, 11, 3, 128)
    w = jnp.pad(w, ((0, 1), (0, 1), (0, 0), (0, 0)))
    w = w.reshape(3, 4, 3, 4, 3, 128).transpose(0, 2, 1, 3, 4, 5)
    w = w.reshape(9, 48, 128)
    return jnp.pad(w, ((0, 0), (0, 80), (0, 0))).reshape(9 * 128, 128)


def kernel(x, b1_w1, b1_b1, b1_w2, b1_b2, b1_w3, b1_b3,
           b2_w1, b2_b1, b2_w2, b2_b2, b2_w3, b2_b3,
           b3_w1, b3_b1, b3_w2, b3_b2, b3_w3, b3_b3,
           b4_w1, b4_b1, b4_w2, b4_b2, b4_w3, b4_b3):
    w11 = _prep_w1_b1(b1_w1)
    w21 = jnp.pad(b2_w1[:2400].reshape(25, 96, 256),
                  ((0, 0), (0, 32), (0, 0))).reshape(25 * 128, 256)
    w31 = b3_w1[:2304]
    w41 = b4_w1[:3456]

    x1 = _space_to_depth4(x)
    x2 = _run_block(x1, w11, b1_b1, b1_w2, b1_b2, b1_w3, b1_b3,
                    hp=57, wp=56, hv=54, kh=3, kw=3, pool="max",
                    oh=31, ow=32, opad=2)
    x3 = _run_block(x2, w21, b2_b1, b2_w2, b2_b2, b2_w3, b2_b3,
                    hp=31, wp=32, hv=26, kh=5, kw=5, pool="max",
                    oh=15, ow=16, opad=1)
    x4 = _run_block(x3, w31, b3_b1, b3_w2, b3_b2, b3_w3, b3_b3,
                    hp=15, wp=16, hv=12, kh=3, kw=3, pool="max",
                    oh=8, ow=8, opad=1)
    out = _run_block(x4, w41, b4_b1, b4_w2, b4_b2, b4_w3, b4_b3,
                     hp=8, wp=8, hv=5, kh=3, kw=3, pool="avg")
    return out[:, 0, :10]


# speedup vs baseline: 1.3290x; 1.0100x over previous
import functools

import jax
import jax.numpy as jnp
from jax.experimental import pallas as pl
from jax.experimental.pallas import tpu as pltpu


def _maxpool_3x3_s2(h, hv, wp):
    c = h.shape[-1]
    po = (hv - 3) // 2 + 1
    g = h.reshape(hv, wp, c)
    a = g[: 2 * po].reshape(po, 2, wp, c).max(axis=1)
    b = g[2 : 2 * po + 2].reshape(po, 2, wp, c)[:, 0]
    rows = jnp.maximum(a, b)
    c1 = rows[:, : 2 * po].reshape(po, po, 2, c).max(axis=2)
    c2 = rows[:, 2 : 2 * po + 2].reshape(po, po, 2, c)[:, :, 0]
    return jnp.maximum(c1, c2)


def _block_body(x_ref, w1_ref, b1_ref, w2_ref, b2_ref, w3_ref, b3_ref, o_ref,
                *, hp, wp, hv, kh, kw, pool, opad):
    cin = x_ref.shape[-1]
    cout = w1_ref.shape[-1]
    x = x_ref[0].reshape(hp * wp, cin)
    l = hv * wp
    patches = jnp.concatenate(
        [x[i * wp + j : i * wp + j + l]
         for i in range(kh) for j in range(kw)], axis=1)
    h = jnp.dot(patches, w1_ref[...], preferred_element_type=jnp.float32)
    h = jnp.maximum(h + b1_ref[...], 0.0).astype(jnp.bfloat16)
    h = jnp.dot(h, w2_ref[...], preferred_element_type=jnp.float32)
    h = jnp.maximum(h + b2_ref[...], 0.0).astype(jnp.bfloat16)
    h = jnp.dot(h, w3_ref[...], preferred_element_type=jnp.float32)
    h = jnp.maximum(h + b3_ref[...], 0.0).astype(jnp.bfloat16)
    if pool == "max":
        po = (hv - 3) // 2 + 1
        o_ref[0] = jnp.zeros(o_ref.shape[1:], o_ref.dtype)
        o_ref[0, opad : opad + po, opad : opad + po, :] = _maxpool_3x3_s2(
            h, hv, wp)
    else:
        g = h.reshape(hv, wp, cout)[:, :hv].astype(jnp.float32)
        o_ref[0] = g.sum(axis=0).sum(axis=0, keepdims=True) / (hv * hv)


def _run_block(x, w1, b1, w2, b2, w3, b3, *, hp, wp, hv, kh, kw, pool,
               oh=0, ow=0, opad=0):
    n, cin = x.shape[0], x.shape[-1]
    cout = w1.shape[-1]
    body = functools.partial(_block_body, hp=hp, wp=wp, hv=hv, kh=kh, kw=kw,
                             pool=pool, opad=opad)
    if pool == "max":
        out_shape = jax.ShapeDtypeStruct((n, oh, ow, cout), jnp.bfloat16)
        out_spec = pl.BlockSpec((1, oh, ow, cout), lambda i: (i, 0, 0, 0))
    else:
        out_shape = jax.ShapeDtypeStruct((n, 1, cout), jnp.float32)
        out_spec = pl.BlockSpec((1, 1, cout), lambda i: (i, 0, 0))
    return pl.pallas_call(
        body,
        out_shape=out_shape,
        grid=(n,),
        in_specs=[
            pl.BlockSpec((1, hp, wp, cin), lambda i: (i, 0, 0, 0)),
            pl.BlockSpec(w1.shape, lambda i: (0, 0)),
            pl.BlockSpec(b1.shape, lambda i: (0, 0)),
            pl.BlockSpec(w2.shape, lambda i: (0, 0)),
            pl.BlockSpec(b2.shape, lambda i: (0, 0)),
            pl.BlockSpec(w3.shape, lambda i: (0, 0)),
            pl.BlockSpec(b3.shape, lambda i: (0, 0)),
        ],
        out_specs=out_spec,
        compiler_params=pltpu.CompilerParams(
            dimension_semantics=("parallel",)),
    )(x, w1, b1, w2, b2, w3, b3)


def _space_to_depth4(x_nchw):
    n = x_nchw.shape[0]
    x = x_nchw.astype(jnp.bfloat16)
    x = x.reshape(n, 3, 56, 4, 56, 4).transpose(0, 2, 4, 3, 5, 1)
    x = x.reshape(n, 56, 56, 48)
    return jnp.pad(x, ((0, 0), (0, 1), (0, 0), (0, 80)))


def _prep_w1_b1(w):
    w = w[:363].reshape(11, 11, 3, 128)
    w = jnp.pad(w, ((0, 1), (0, 1), (0, 0), (0, 0)))
    w = w.reshape(3, 4, 3, 4, 3, 128).transpose(0, 2, 1, 3, 4, 5)
    w = w.reshape(9, 48, 128)
    return jnp.pad(w, ((0, 0), (0, 80), (0, 0))).reshape(9 * 128, 128)


def kernel(x, b1_w1, b1_b1, b1_w2, b1_b2, b1_w3, b1_b3,
           b2_w1, b2_b1, b2_w2, b2_b2, b2_w3, b2_b3,
           b3_w1, b3_b1, b3_w2, b3_b2, b3_w3, b3_b3,
           b4_w1, b4_b1, b4_w2, b4_b2, b4_w3, b4_b3):
    w11 = _prep_w1_b1(b1_w1)
    w21 = jnp.pad(b2_w1[:2400].reshape(25, 96, 256),
                  ((0, 0), (0, 32), (0, 0))).reshape(25 * 128, 256)
    w31 = b3_w1[:2304]
    w41 = b4_w1[:3456]

    n = x.shape[0]
    outs = []
    for xh in (x[: n // 2], x[n // 2 :]):
        x1 = _space_to_depth4(xh)
        x2 = _run_block(x1, w11, b1_b1, b1_w2, b1_b2, b1_w3, b1_b3,
                        hp=57, wp=56, hv=54, kh=3, kw=3, pool="max",
                        oh=31, ow=32, opad=2)
        x3 = _run_block(x2, w21, b2_b1, b2_w2, b2_b2, b2_w3, b2_b3,
                        hp=31, wp=32, hv=26, kh=5, kw=5, pool="max",
                        oh=15, ow=16, opad=1)
        x4 = _run_block(x3, w31, b3_b1, b3_w2, b3_b2, b3_w3, b3_b3,
                        hp=15, wp=16, hv=12, kh=3, kw=3, pool="max",
                        oh=8, ow=8, opad=1)
        outs.append(_run_block(x4, w41, b4_b1, b4_w2, b4_b2, b4_w3, b4_b3,
                               hp=8, wp=8, hv=5, kh=3, kw=3, pool="avg"))
    return jnp.concatenate(outs, axis=0)[:, 0, :10]


# speedup vs baseline: 1.3626x; 1.0253x over previous
import functools

import jax
import jax.numpy as jnp
from jax.experimental import pallas as pl
from jax.experimental.pallas import tpu as pltpu


def _maxpool_3x3_s2(h, hv, wp):
    c = h.shape[-1]
    po = (hv - 3) // 2 + 1
    g = h.reshape(hv, wp, c)
    a = g[: 2 * po].reshape(po, 2, wp, c).max(axis=1)
    b = g[2 : 2 * po + 2].reshape(po, 2, wp, c)[:, 0]
    rows = jnp.maximum(a, b)
    c1 = rows[:, : 2 * po].reshape(po, po, 2, c).max(axis=2)
    c2 = rows[:, 2 : 2 * po + 2].reshape(po, po, 2, c)[:, :, 0]
    return jnp.maximum(c1, c2)


def _block_body(x_ref, w1_ref, b1_ref, w2_ref, b2_ref, w3_ref, b3_ref, o_ref,
                *, hp, wp, hv, kh, kw, pool, opad):
    cin = x_ref.shape[-1]
    cout = w1_ref.shape[-1]
    x = x_ref[0].reshape(hp * wp, cin)
    l = hv * wp
    patches = jnp.concatenate(
        [x[i * wp + j : i * wp + j + l]
         for i in range(kh) for j in range(kw)], axis=1)
    h = jnp.dot(patches, w1_ref[...], preferred_element_type=jnp.float32)
    h = jnp.maximum(h + b1_ref[...], 0.0).astype(jnp.bfloat16)
    h = jnp.dot(h, w2_ref[...], preferred_element_type=jnp.float32)
    h = jnp.maximum(h + b2_ref[...], 0.0).astype(jnp.bfloat16)
    h = jnp.dot(h, w3_ref[...], preferred_element_type=jnp.float32)
    h = jnp.maximum(h + b3_ref[...], 0.0).astype(jnp.bfloat16)
    if pool == "max":
        po = (hv - 3) // 2 + 1
        o_ref[0] = jnp.zeros(o_ref.shape[1:], o_ref.dtype)
        o_ref[0, opad : opad + po, opad : opad + po, :] = _maxpool_3x3_s2(
            h, hv, wp)
    else:
        g = h.reshape(hv, wp, cout)[:, :hv].astype(jnp.float32)
        o_ref[0] = g.sum(axis=0).sum(axis=0, keepdims=True) / (hv * hv)


def _run_block(x, w1, b1, w2, b2, w3, b3, *, hp, wp, hv, kh, kw, pool,
               oh=0, ow=0, opad=0):
    n, cin = x.shape[0], x.shape[-1]
    cout = w1.shape[-1]
    body = functools.partial(_block_body, hp=hp, wp=wp, hv=hv, kh=kh, kw=kw,
                             pool=pool, opad=opad)
    if pool == "max":
        out_shape = jax.ShapeDtypeStruct((n, oh, ow, cout), jnp.bfloat16)
        out_spec = pl.BlockSpec((1, oh, ow, cout), lambda i: (i, 0, 0, 0))
    else:
        out_shape = jax.ShapeDtypeStruct((n, 1, cout), jnp.float32)
        out_spec = pl.BlockSpec((1, 1, cout), lambda i: (i, 0, 0))
    return pl.pallas_call(
        body,
        out_shape=out_shape,
        grid=(n,),
        in_specs=[
            pl.BlockSpec((1, hp, wp, cin), lambda i: (i, 0, 0, 0)),
            pl.BlockSpec(w1.shape, lambda i: (0, 0)),
            pl.BlockSpec(b1.shape, lambda i: (0, 0)),
            pl.BlockSpec(w2.shape, lambda i: (0, 0)),
            pl.BlockSpec(b2.shape, lambda i: (0, 0)),
            pl.BlockSpec(w3.shape, lambda i: (0, 0)),
            pl.BlockSpec(b3.shape, lambda i: (0, 0)),
        ],
        out_specs=out_spec,
        compiler_params=pltpu.CompilerParams(
            dimension_semantics=("parallel",)),
    )(x, w1, b1, w2, b2, w3, b3)


def _space_to_depth4(x_nchw):
    n = x_nchw.shape[0]
    x = x_nchw.astype(jnp.bfloat16)
    x = x.reshape(n, 3, 56, 4, 56, 4).transpose(0, 2, 4, 3, 5, 1)
    x = x.reshape(n, 56, 56, 48)
    return jnp.pad(x, ((0, 0), (0, 1), (0, 0), (0, 16)))


def _prep_w1_b1(w):
    w = w[:363].reshape(11, 11, 3, 128)
    w = jnp.pad(w, ((0, 1), (0, 1), (0, 0), (0, 0)))
    w = w.reshape(3, 4, 3, 4, 3, 128).transpose(0, 2, 1, 3, 4, 5)
    w = w.reshape(9, 48, 128)
    return jnp.pad(w, ((0, 0), (0, 16), (0, 0))).reshape(9 * 64, 128)


def kernel(x, b1_w1, b1_b1, b1_w2, b1_b2, b1_w3, b1_b3,
           b2_w1, b2_b1, b2_w2, b2_b2, b2_w3, b2_b3,
           b3_w1, b3_b1, b3_w2, b3_b2, b3_w3, b3_b3,
           b4_w1, b4_b1, b4_w2, b4_b2, b4_w3, b4_b3):
    w11 = _prep_w1_b1(b1_w1)
    w21 = jnp.pad(b2_w1[:2400].reshape(25, 96, 256),
                  ((0, 0), (0, 32), (0, 0))).reshape(25 * 128, 256)
    w31 = b3_w1[:2304]
    w41 = b4_w1[:3456]

    n = x.shape[0]
    outs = []
    for xh in (x[: n // 2], x[n // 2 :]):
        x1 = _space_to_depth4(xh)
        x2 = _run_block(x1, w11, b1_b1, b1_w2, b1_b2, b1_w3, b1_b3,
                        hp=57, wp=56, hv=54, kh=3, kw=3, pool="max",
                        oh=31, ow=32, opad=2)
        x3 = _run_block(x2, w21, b2_b1, b2_w2, b2_b2, b2_w3, b2_b3,
                        hp=31, wp=32, hv=26, kh=5, kw=5, pool="max",
                        oh=15, ow=16, opad=1)
        x4 = _run_block(x3, w31, b3_b1, b3_w2, b3_b2, b3_w3, b3_b3,
                        hp=15, wp=16, hv=12, kh=3, kw=3, pool="max",
                        oh=8, ow=8, opad=1)
        outs.append(_run_block(x4, w41, b4_b1, b4_w2, b4_b2, b4_w3, b4_b3,
                               hp=8, wp=8, hv=5, kh=3, kw=3, pool="avg"))
    return jnp.concatenate(outs, axis=0)[:, 0, :10]
